# Initial kernel scaffold; baseline (speedup 1.0000x reference)
import jax
import jax.numpy as jnp
from jax.experimental import pallas as pl


def kernel(x, w1, b1, w2, b2):
    raise NotImplementedError("write your pallas kernel here")


# speedup vs baseline: 3.2239x; 3.2239x over previous
import functools

import jax
import jax.numpy as jnp
from jax.experimental import pallas as pl
from jax.experimental.pallas import tpu as pltpu

N_STATES = 8
N_ACTIONS = 4
HIDDEN = 50
HIDDEN_P = 128
TILE_B = 4096


def _mlp_kernel(x_ref, w1_ref, b1_ref, w2_ref, b2_ref, o_ref):
    h = jnp.dot(x_ref[...].astype(jnp.bfloat16), w1_ref[...],
                preferred_element_type=jnp.float32)
    h = jnp.maximum(h + b1_ref[...], 0.0).astype(jnp.bfloat16)
    q = jnp.dot(h, w2_ref[...], preferred_element_type=jnp.float32)
    o_ref[...] = q[:, :N_ACTIONS] + b2_ref[...]


@functools.partial(jax.jit, static_argnames=())
def kernel(x, w1, b1, w2, b2):
    B = x.shape[0]

    w1_p = (jnp.zeros((N_STATES, HIDDEN_P), jnp.float32)
            .at[:, :HIDDEN].set(w1).astype(jnp.bfloat16))
    b1_p = jnp.zeros((1, HIDDEN_P), jnp.float32).at[:, :HIDDEN].set(b1)
    w2_p = (jnp.zeros((HIDDEN_P, HIDDEN_P), jnp.float32)
            .at[:HIDDEN, :N_ACTIONS].set(w2).astype(jnp.bfloat16))

    b_pad = -(-B // TILE_B) * TILE_B
    x_p = x if b_pad == B else jnp.zeros((b_pad, N_STATES), jnp.float32).at[:B].set(x)

    flops = 2 * b_pad * (N_STATES * HIDDEN_P + HIDDEN_P * HIDDEN_P)
    bytes_accessed = 4 * b_pad * N_STATES + 4 * b_pad * N_ACTIONS + 2 * (
        N_STATES * HIDDEN_P + HIDDEN_P + HIDDEN_P * HIDDEN_P) + 4 * N_ACTIONS

    out = pl.pallas_call(
        _mlp_kernel,
        out_shape=jax.ShapeDtypeStruct((b_pad, N_ACTIONS), jnp.float32),
        grid=(b_pad // TILE_B,),
        in_specs=[
            pl.BlockSpec((TILE_B, N_STATES), lambda i: (i, 0)),
            pl.BlockSpec((N_STATES, HIDDEN_P), lambda i: (0, 0)),
            pl.BlockSpec((1, HIDDEN_P), lambda i: (0, 0)),
            pl.BlockSpec((HIDDEN_P, HIDDEN_P), lambda i: (0, 0)),
            pl.BlockSpec((1, N_ACTIONS), lambda i: (0, 0)),
        ],
        out_specs=pl.BlockSpec((TILE_B, N_ACTIONS), lambda i: (i, 0)),
        compiler_params=pltpu.CompilerParams(
            dimension_semantics=("parallel",)),
        cost_estimate=pl.CostEstimate(flops=flops, transcendentals=0,
                                      bytes_accessed=bytes_accessed),
    )(x_p, w1_p, b1_p, w2_p, b2)

    return out[:B]


# speedup vs baseline: 3.5722x; 1.1080x over previous
import functools

import jax
import jax.numpy as jnp
from jax.experimental import pallas as pl
from jax.experimental.pallas import tpu as pltpu

N_STATES = 8
N_ACTIONS = 4
HIDDEN = 50
HIDDEN_P = 128
TILE_B = 16384


def _mlp_kernel(x_ref, w1_ref, b1_ref, w2_ref, b2_ref, o_ref):
    h = jnp.dot(x_ref[...].astype(jnp.bfloat16), w1_ref[...],
                preferred_element_type=jnp.float32)
    h = jnp.maximum(h + b1_ref[...], 0.0).astype(jnp.bfloat16)
    q = jnp.dot(h, w2_ref[...], preferred_element_type=jnp.float32)
    o_ref[...] = q[:, :N_ACTIONS] + b2_ref[...]


@functools.partial(jax.jit, static_argnames=())
def kernel(x, w1, b1, w2, b2):
    B = x.shape[0]

    w1_p = (jnp.zeros((N_STATES, HIDDEN_P), jnp.float32)
            .at[:, :HIDDEN].set(w1).astype(jnp.bfloat16))
    b1_p = jnp.zeros((1, HIDDEN_P), jnp.float32).at[:, :HIDDEN].set(b1)
    w2_p = (jnp.zeros((HIDDEN_P, HIDDEN_P), jnp.float32)
            .at[:HIDDEN, :N_ACTIONS].set(w2).astype(jnp.bfloat16))

    b_pad = -(-B // TILE_B) * TILE_B
    x_p = x if b_pad == B else jnp.zeros((b_pad, N_STATES), jnp.float32).at[:B].set(x)

    flops = 2 * b_pad * (N_STATES * HIDDEN_P + HIDDEN_P * HIDDEN_P)
    bytes_accessed = 4 * b_pad * N_STATES + 4 * b_pad * N_ACTIONS + 2 * (
        N_STATES * HIDDEN_P + HIDDEN_P + HIDDEN_P * HIDDEN_P) + 4 * N_ACTIONS

    out = pl.pallas_call(
        _mlp_kernel,
        out_shape=jax.ShapeDtypeStruct((b_pad, N_ACTIONS), jnp.float32),
        grid=(b_pad // TILE_B,),
        in_specs=[
            pl.BlockSpec((TILE_B, N_STATES), lambda i: (i, 0)),
            pl.BlockSpec((N_STATES, HIDDEN_P), lambda i: (0, 0)),
            pl.BlockSpec((1, HIDDEN_P), lambda i: (0, 0)),
            pl.BlockSpec((HIDDEN_P, HIDDEN_P), lambda i: (0, 0)),
            pl.BlockSpec((1, N_ACTIONS), lambda i: (0, 0)),
        ],
        out_specs=pl.BlockSpec((TILE_B, N_ACTIONS), lambda i: (i, 0)),
        compiler_params=pltpu.CompilerParams(
            dimension_semantics=("parallel",)),
        cost_estimate=pl.CostEstimate(flops=flops, transcendentals=0,
                                      bytes_accessed=bytes_accessed),
    )(x_p, w1_p, b1_p, w2_p, b2)

    return out[:B]
